# Initial kernel scaffold; baseline (speedup 1.0000x reference)
import functools

import jax
import jax.numpy as jnp
from jax import lax
from jax.experimental import pallas as pl
from jax.experimental.pallas import tpu as pltpu
from jax.experimental.pallas import tpu_sc as plsc


def _transform_body(S, K, M, f_ref, invl_ref, gt_ref, t0_ref, t1_ref, t2_ref):
    outs = (t0_ref, t1_ref, t2_ref)
    for s in range(S):
        f = f_ref[s * K:(s + 1) * K, :]
        scaled = []
        for j in range(3):
            col = (f[:, 0:1] * invl_ref[s, 0:1, j:j + 1]
                   + f[:, 1:2] * invl_ref[s, 1:2, j:j + 1]
                   + f[:, 2:3] * invl_ref[s, 2:3, j:j + 1])
            scaled.append(col)
        for i in range(3):
            acc = None
            for j in range(3):
                g = gt_ref[j, i, s, :].reshape(1, M)
                term = scaled[j] * g
                acc = term if acc is None else acc + term
            outs[i][s * K:(s + 1) * K, :] = acc


def _make_transform(B, K, M, S, interpret=False):
    grid = (B // S,)
    return pl.pallas_call(
        functools.partial(_transform_body, S, K, M),
        grid=grid,
        in_specs=[
            pl.BlockSpec((S * K, 3), lambda b: (b, 0)),
            pl.BlockSpec((S, 3, 3), lambda b: (b, 0, 0)),
            pl.BlockSpec((3, 3, S, M), lambda b: (0, 0, b, 0)),
        ],
        out_specs=[
            pl.BlockSpec((S * K, M), lambda b: (b, 0)),
            pl.BlockSpec((S * K, M), lambda b: (b, 0)),
            pl.BlockSpec((S * K, M), lambda b: (b, 0)),
        ],
        out_shape=[jax.ShapeDtypeStruct((B * K, M), jnp.float32)] * 3,
        interpret=interpret,
    )


def _make_scatter(NA, ROWS):
    RPC = ROWS // 2
    RPT = RPC // 16
    CH = 16
    NCH = RPT // CH
    Z = NA // 16

    mesh = plsc.VectorSubcoreMesh(core_axis_name="c", subcore_axis_name="s")

    @functools.partial(
        pl.kernel,
        mesh=mesh,
        out_type=jax.ShapeDtypeStruct((2, 3, NA), jnp.float32),
        scratch_types=[
            pltpu.VMEM((CH, 128), jnp.int32),
            pltpu.VMEM((CH, 128), jnp.float32),
            pltpu.VMEM((CH, 128), jnp.float32),
            pltpu.VMEM((CH, 128), jnp.float32),
            pltpu.VMEM_SHARED((NA,), jnp.float32),
            pltpu.VMEM_SHARED((NA,), jnp.float32),
            pltpu.VMEM_SHARED((NA,), jnp.float32),
        ],
    )
    def scatter(idx_hbm, v0_hbm, v1_hbm, v2_hbm, zeros_hbm, out_hbm,
                idxb, b0, b1, b2, acc0, acc1, acc2):
        cid = lax.axis_index("c")
        sid = lax.axis_index("s")
        accs = (acc0, acc1, acc2)
        for i in range(3):
            pltpu.sync_copy(zeros_hbm.at[pl.ds(sid * Z, Z)],
                            accs[i].at[pl.ds(sid * Z, Z)])
        plsc.subcore_barrier()

        base = cid * RPC + sid * RPT

        def chunk_body(c, carry):
            row0 = base + c * CH
            pltpu.sync_copy(idx_hbm.at[pl.ds(row0, CH)], idxb)
            pltpu.sync_copy(v0_hbm.at[pl.ds(row0, CH)], b0)
            pltpu.sync_copy(v1_hbm.at[pl.ds(row0, CH)], b1)
            pltpu.sync_copy(v2_hbm.at[pl.ds(row0, CH)], b2)
            for j in range(CH):
                idxrow = idxb.at[j]
                pltpu.sync_copy(b0.at[j], acc0.at[idxrow], add=True)
                pltpu.sync_copy(b1.at[j], acc1.at[idxrow], add=True)
                pltpu.sync_copy(b2.at[j], acc2.at[idxrow], add=True)
            return carry

        lax.fori_loop(0, NCH, chunk_body, 0)
        plsc.subcore_barrier()
        for i in range(3):
            pltpu.sync_copy(accs[i].at[pl.ds(sid * Z, Z)],
                            out_hbm.at[cid, i, pl.ds(sid * Z, Z)])

    return scatter


def _final_body(S, K, p_ref, lat_ref, den_ref, o_ref):
    accs = []
    for s in range(S):
        accs.append([
            (p_ref[0, j, s * K:(s + 1) * K]
             + p_ref[1, j, s * K:(s + 1) * K]).reshape(1, K)
            for j in range(3)
        ])
    rows = []
    for i in range(3):
        segs = []
        for s in range(S):
            den = den_ref[0, 0:1, s:s + 1]
            c = (accs[s][0] * lat_ref[s, 0:1, i:i + 1]
                 + accs[s][1] * lat_ref[s, 1:2, i:i + 1]
                 + accs[s][2] * lat_ref[s, 2:3, i:i + 1]) / den
            segs.append(c)
        rows.append(jnp.concatenate(segs, axis=1))
    blk = jnp.concatenate(rows, axis=0)
    o_ref[...] = blk.T


def _make_final(B, K, S, interpret=False):
    NA = B * K
    grid = (B // S,)
    return pl.pallas_call(
        functools.partial(_final_body, S, K),
        grid=grid,
        in_specs=[
            pl.BlockSpec((2, 3, S * K), lambda b: (0, 0, b)),
            pl.BlockSpec((S, 3, 3), lambda b: (b, 0, 0)),
            pl.BlockSpec((1, 1, S), lambda b: (b, 0, 0)),
        ],
        out_specs=pl.BlockSpec((S * K, 3), lambda b: (b, 0)),
        out_shape=jax.ShapeDtypeStruct((NA, 3), jnp.float32),
        interpret=interpret,
    )


def kernel(lattices, inv_lattices, forces, num_atoms, general_ops, symm_map,
           num_general_ops):
    B = lattices.shape[0]
    na = forces.shape[0]
    M = general_ops.shape[1]
    K = na // B
    S = 8
    ROWS = (na * M) // 128

    gt = jnp.transpose(general_ops[:, :, :3, :3], (3, 2, 0, 1))
    t0, t1, t2 = _make_transform(B, K, M, S)(forces, inv_lattices, gt)

    idx2 = symm_map.reshape(ROWS, 128)
    v0 = t0.reshape(ROWS, 128)
    v1 = t1.reshape(ROWS, 128)
    v2 = t2.reshape(ROWS, 128)
    zeros = jnp.zeros((na,), jnp.float32)
    partials = _make_scatter(na, ROWS)(idx2, v0, v1, v2, zeros)

    den3 = num_general_ops.astype(jnp.float32).reshape(B // S, 1, S)
    return _make_final(B, K, S)(partials, lattices, den3)


# speedup vs baseline: 18.5495x; 18.5495x over previous
import functools

import jax
import jax.numpy as jnp
from jax import lax
from jax.experimental import pallas as pl
from jax.experimental.pallas import tpu as pltpu
from jax.experimental.pallas import tpu_sc as plsc


def _transform_body(S, K, M, f_ref, invl_ref, gt_ref, t0_ref, t1_ref, t2_ref):
    outs = (t0_ref, t1_ref, t2_ref)
    for s in range(S):
        f = f_ref[s * K:(s + 1) * K, :]
        scaled = []
        for j in range(3):
            col = (f[:, 0:1] * invl_ref[s, 0:1, j:j + 1]
                   + f[:, 1:2] * invl_ref[s, 1:2, j:j + 1]
                   + f[:, 2:3] * invl_ref[s, 2:3, j:j + 1])
            scaled.append(col)
        for i in range(3):
            acc = None
            for j in range(3):
                g = gt_ref[j, i, s, :].reshape(1, M)
                term = scaled[j] * g
                acc = term if acc is None else acc + term
            outs[i][s * K:(s + 1) * K, :] = acc


def _make_transform(B, K, M, S, interpret=False):
    grid = (B // S,)
    return pl.pallas_call(
        functools.partial(_transform_body, S, K, M),
        grid=grid,
        in_specs=[
            pl.BlockSpec((S * K, 3), lambda b: (b, 0)),
            pl.BlockSpec((S, 3, 3), lambda b: (b, 0, 0)),
            pl.BlockSpec((3, 3, S, M), lambda b: (0, 0, b, 0)),
        ],
        out_specs=[
            pl.BlockSpec((S * K, M), lambda b: (b, 0)),
            pl.BlockSpec((S * K, M), lambda b: (b, 0)),
            pl.BlockSpec((S * K, M), lambda b: (b, 0)),
        ],
        out_shape=[jax.ShapeDtypeStruct((B * K, M), jnp.float32)] * 3,
        interpret=interpret,
    )


def _make_scatter(NA, ROWS):
    RPC = ROWS // 2
    RPT = RPC // 16
    CH = 16
    NCH = RPT // CH
    Z = NA // 16

    mesh = plsc.VectorSubcoreMesh(core_axis_name="c", subcore_axis_name="s")

    @functools.partial(
        pl.kernel,
        mesh=mesh,
        out_type=jax.ShapeDtypeStruct((6 * NA,), jnp.float32),
        scratch_types=[
            pltpu.VMEM((CH, 128), jnp.int32),
            pltpu.VMEM((CH, 128), jnp.float32),
            pltpu.VMEM((CH, 128), jnp.float32),
            pltpu.VMEM((CH, 128), jnp.float32),
            pltpu.VMEM_SHARED((NA,), jnp.float32),
            pltpu.VMEM_SHARED((NA,), jnp.float32),
            pltpu.VMEM_SHARED((NA,), jnp.float32),
        ],
    )
    def scatter(idx_hbm, v0_hbm, v1_hbm, v2_hbm, zeros_hbm, out_hbm,
                idxb, b0, b1, b2, acc0, acc1, acc2):
        cid = lax.axis_index("c")
        sid = lax.axis_index("s")
        accs = (acc0, acc1, acc2)
        for i in range(3):
            pltpu.sync_copy(zeros_hbm.at[pl.ds(sid * Z, Z)],
                            accs[i].at[pl.ds(sid * Z, Z)])
        plsc.subcore_barrier()

        base = cid * RPC + sid * RPT

        def chunk_body(c, carry):
            row0 = base + c * CH
            pltpu.sync_copy(idx_hbm.at[pl.ds(row0, CH)], idxb)
            pltpu.sync_copy(v0_hbm.at[pl.ds(row0, CH)], b0)
            pltpu.sync_copy(v1_hbm.at[pl.ds(row0, CH)], b1)
            pltpu.sync_copy(v2_hbm.at[pl.ds(row0, CH)], b2)
            for j in range(CH):
                idxrow = idxb.at[j]
                pltpu.sync_copy(b0.at[j], acc0.at[idxrow], add=True)
                pltpu.sync_copy(b1.at[j], acc1.at[idxrow], add=True)
                pltpu.sync_copy(b2.at[j], acc2.at[idxrow], add=True)
            return carry

        lax.fori_loop(0, NCH, chunk_body, 0)
        plsc.subcore_barrier()
        for i in range(3):
            pltpu.sync_copy(accs[i].at[pl.ds(sid * Z, Z)],
                            out_hbm.at[pl.ds((cid * 3 + i) * NA + sid * Z, Z)])

    return scatter


def _final_body(S, K, p00, p01, p02, p10, p11, p12, lat_ref, den_ref, o_ref):
    pc = ((p00, p01, p02), (p10, p11, p12))
    accs = []
    for s in range(S):
        accs.append([
            (pc[0][j][s * K:(s + 1) * K]
             + pc[1][j][s * K:(s + 1) * K]).reshape(1, K)
            for j in range(3)
        ])
    rows = []
    for i in range(3):
        segs = []
        for s in range(S):
            den = den_ref[0, 0:1, s:s + 1]
            c = (accs[s][0] * lat_ref[s, 0:1, i:i + 1]
                 + accs[s][1] * lat_ref[s, 1:2, i:i + 1]
                 + accs[s][2] * lat_ref[s, 2:3, i:i + 1]) / den
            segs.append(c)
        rows.append(jnp.concatenate(segs, axis=1))
    blk = jnp.concatenate(rows, axis=0)
    o_ref[...] = blk.T


def _make_final(B, K, S, interpret=False):
    NA = B * K
    grid = (B // S,)
    return pl.pallas_call(
        functools.partial(_final_body, S, K),
        grid=grid,
        in_specs=[
            pl.BlockSpec((S * K,), lambda b, o=off: (o + b,))
            for off in [q * NA // (S * K) for q in range(6)]
        ] + [
            pl.BlockSpec((S, 3, 3), lambda b: (b, 0, 0)),
            pl.BlockSpec((1, 1, S), lambda b: (b, 0, 0)),
        ],
        out_specs=pl.BlockSpec((S * K, 3), lambda b: (b, 0)),
        out_shape=jax.ShapeDtypeStruct((NA, 3), jnp.float32),
        interpret=interpret,
    )


def kernel(lattices, inv_lattices, forces, num_atoms, general_ops, symm_map,
           num_general_ops):
    B = lattices.shape[0]
    na = forces.shape[0]
    M = general_ops.shape[1]
    K = na // B
    S = 8
    ROWS = (na * M) // 128

    gt = jnp.transpose(general_ops[:, :, :3, :3], (3, 2, 0, 1))
    t0, t1, t2 = _make_transform(B, K, M, S)(forces, inv_lattices, gt)

    idx2 = symm_map.reshape(ROWS, 128)
    v0 = t0.reshape(ROWS, 128)
    v1 = t1.reshape(ROWS, 128)
    v2 = t2.reshape(ROWS, 128)
    zeros = jnp.zeros((na,), jnp.float32)
    p = _make_scatter(na, ROWS)(idx2, v0, v1, v2, zeros)

    den3 = num_general_ops.astype(jnp.float32).reshape(B // S, 1, S)
    return _make_final(B, K, S)(p, p, p, p, p, p, lattices, den3)
